# Initial kernel scaffold; baseline (speedup 1.0000x reference)
import jax
import jax.numpy as jnp
from jax.experimental import pallas as pl


def kernel(values_0, values_1, values_2, values_3, weight_0, weight_1, weight_2, weight_3):
    raise NotImplementedError("write your pallas kernel here")


# speedup vs baseline: 7.2102x; 7.2102x over previous
import functools

import jax
import jax.numpy as jnp
from jax import lax
from jax.experimental import pallas as pl
from jax.experimental.pallas import tpu as pltpu
from jax.experimental.pallas import tpu_sc as plsc

_NUM_TABLES = 4
_V = 100000
_D = 128
_B = 204800

_NC = 2
_NS = 16
_NW = _NC * _NS
_B_PER_W = _B // _NW
_CHUNK = 128
_N_CHUNKS = _B_PER_W // _CHUNK


def _grouped_embedding_body(v0, v1, v2, v3, w0, w1, w2, w3, out,
                            idx_v, rows_v, gsem):
    wid = lax.axis_index("s") * _NC + lax.axis_index("c")
    base = wid * _B_PER_W
    values = (v0, v1, v2, v3)
    weights = (w0, w1, w2, w3)
    for t in range(_NUM_TABLES):
        pltpu.sync_copy(values[t].at[pl.ds(base, _B_PER_W)], idx_v)

        def chunk_body(c, carry, t=t):
            off = c * _CHUNK
            pltpu.async_copy(
                weights[t].at[idx_v.at[pl.ds(off, _CHUNK)]],
                rows_v, gsem).wait()
            pltpu.sync_copy(rows_v,
                            out.at[pl.ds(t * _B + base + off, _CHUNK)])
            return carry

        lax.fori_loop(0, _N_CHUNKS, chunk_body, 0)


@functools.partial(
    pl.kernel,
    mesh=plsc.VectorSubcoreMesh(core_axis_name="c", subcore_axis_name="s"),
    out_type=jax.ShapeDtypeStruct((_NUM_TABLES * _B, _D), jnp.float32),
    scratch_types=[
        pltpu.VMEM((_B_PER_W,), jnp.int32),
        pltpu.VMEM((_CHUNK, _D), jnp.float32),
        pltpu.SemaphoreType.DMA,
    ],
)
def _grouped_embedding(*refs):
    _grouped_embedding_body(*refs)


def kernel(values_0, values_1, values_2, values_3,
           weight_0, weight_1, weight_2, weight_3):
    return _grouped_embedding(values_0, values_1, values_2, values_3,
                              weight_0, weight_1, weight_2, weight_3)


# speedup vs baseline: 8.6593x; 1.2010x over previous
import functools

import jax
import jax.numpy as jnp
from jax import lax
from jax.experimental import pallas as pl
from jax.experimental.pallas import tpu as pltpu
from jax.experimental.pallas import tpu_sc as plsc

_NUM_TABLES = 4
_V = 100000
_D = 128
_B = 204800

_NC = 2
_NS = 16
_NW = _NC * _NS
_B_PER_W = _B // _NW
_CHUNK = 128
_N_CHUNKS = _B_PER_W // _CHUNK


def _grouped_embedding_body(v0, v1, v2, v3, w0, w1, w2, w3, out,
                            idx_v, rows0, rows1, gsem0, gsem1, ssem0, ssem1):
    wid = lax.axis_index("s") * _NC + lax.axis_index("c")
    base = wid * _B_PER_W
    values = (v0, v1, v2, v3)
    weights = (w0, w1, w2, w3)
    rows = (rows0, rows1)
    gsem = (gsem0, gsem1)
    ssem = (ssem0, ssem1)

    def step(t, c, j, first):
        off = c * _CHUNK
        if not first:
            pltpu.make_async_copy(
                rows[j], out.at[pl.ds(0, _CHUNK)], ssem[j]).wait()
        pltpu.async_copy(
            weights[t].at[idx_v.at[pl.ds(off, _CHUNK)]],
            rows[j], gsem[j]).wait()
        pltpu.async_copy(rows[j],
                         out.at[pl.ds(t * _B + base + off, _CHUNK)],
                         ssem[j])

    for t in range(_NUM_TABLES):
        pltpu.sync_copy(values[t].at[pl.ds(base, _B_PER_W)], idx_v)
        step(t, 0, 0, first=(t == 0))
        step(t, 1, 1, first=(t == 0))

        def group_body(g, carry, t=t):
            step(t, 2 * g + 0, 0, first=False)
            step(t, 2 * g + 1, 1, first=False)
            return carry

        lax.fori_loop(1, _N_CHUNKS // 2, group_body, 0)
    for j in range(2):
        pltpu.make_async_copy(rows[j], out.at[pl.ds(0, _CHUNK)],
                              ssem[j]).wait()


@functools.partial(
    pl.kernel,
    mesh=plsc.VectorSubcoreMesh(core_axis_name="c", subcore_axis_name="s"),
    out_type=jax.ShapeDtypeStruct((_NUM_TABLES * _B, _D), jnp.float32),
    scratch_types=[
        pltpu.VMEM((_B_PER_W,), jnp.int32),
        pltpu.VMEM((_CHUNK, _D), jnp.float32),
        pltpu.VMEM((_CHUNK, _D), jnp.float32),
        pltpu.SemaphoreType.DMA,
        pltpu.SemaphoreType.DMA,
        pltpu.SemaphoreType.DMA,
        pltpu.SemaphoreType.DMA,
    ],
)
def _grouped_embedding(*refs):
    _grouped_embedding_body(*refs)


def kernel(values_0, values_1, values_2, values_3,
           weight_0, weight_1, weight_2, weight_3):
    return _grouped_embedding(values_0, values_1, values_2, values_3,
                              weight_0, weight_1, weight_2, weight_3)


# speedup vs baseline: 10.5171x; 1.2145x over previous
import functools

import jax
import jax.numpy as jnp
from jax import lax
from jax.experimental import pallas as pl
from jax.experimental.pallas import tpu as pltpu
from jax.experimental.pallas import tpu_sc as plsc

_NUM_TABLES = 4
_V = 100000
_D = 128
_B = 204800

_NC = 2
_NS = 16
_NW = _NC * _NS
_B_PER_W = _B // _NW
_CHUNK = 128
_NCH = _B_PER_W // _CHUNK
_NBUF = 4


def _grouped_embedding_body(v0, v1, v2, v3, w0, w1, w2, w3, out,
                            idx_all, r0, r1, r2, r3,
                            g0, g1, g2, g3, s0, s1, s2, s3):
    wid = lax.axis_index("s") * _NC + lax.axis_index("c")
    base = wid * _B_PER_W
    values = (v0, v1, v2, v3)
    weights = (w0, w1, w2, w3)
    rows = (r0, r1, r2, r3)
    gsem = (g0, g1, g2, g3)
    ssem = (s0, s1, s2, s3)

    for t in range(_NUM_TABLES):
        pltpu.sync_copy(values[t].at[pl.ds(base, _B_PER_W)], idx_all.at[t])

    def issue_gather(t, off, b, reclaim=True):
        if reclaim:
            pltpu.make_async_copy(rows[b], out.at[pl.ds(0, _CHUNK)],
                                  ssem[b]).wait()
        pltpu.async_copy(
            weights[t].at[idx_all.at[t, pl.ds(off, _CHUNK)]],
            rows[b], gsem[b])

    def retire_writeout(t, off, b):
        pltpu.make_async_copy(weights[0].at[pl.ds(0, _CHUNK)],
                              rows[b], gsem[b]).wait()
        pltpu.async_copy(rows[b],
                         out.at[pl.ds(t * _B + base + off, _CHUNK)],
                         ssem[b])

    def full_step(t_out, off_out, b_out, t_g, off_g, b_g, reclaim=True):
        issue_gather(t_g, off_g, b_g, reclaim)
        retire_writeout(t_out, off_out, b_out)

    issue_gather(0, 0 * _CHUNK, 0, reclaim=False)
    issue_gather(0, 1 * _CHUNK, 1, reclaim=False)
    full_step(0, 0 * _CHUNK, 0, 0, 2 * _CHUNK, 2, reclaim=False)
    full_step(0, 1 * _CHUNK, 1, 0, 3 * _CHUNK, 3, reclaim=False)

    def body0(g, carry):
        for j in range(4):
            rel = 4 * g + j + 2
            off_out = rel * _CHUNK
            full_step(0, off_out, (2 + j) % 4,
                      0, off_out + 2 * _CHUNK, j % 4)
        return carry

    lax.fori_loop(0, 11, body0, 0)

    for s in (46, 47, 48, 49):
        sg = s + 2
        tg, cg = divmod(sg, _NCH)
        full_step(0, s * _CHUNK, s % 4, tg, cg * _CHUNK, sg % 4)

    for t in range(1, _NUM_TABLES):
        s0_ = _NCH * t

        def body(g, carry, t=t, s0_=s0_):
            for j in range(4):
                rel = 4 * g + j
                off_out = rel * _CHUNK
                full_step(t, off_out, (s0_ + j) % 4,
                          t, off_out + 2 * _CHUNK, (s0_ + j + 2) % 4)
            return carry

        lax.fori_loop(0, 12, body, 0)

        for s_rel in (48, 49):
            s = s0_ + s_rel
            if t < _NUM_TABLES - 1:
                sg = s + 2
                tg, cg = divmod(sg, _NCH)
                full_step(t, s_rel * _CHUNK, s % 4, tg, cg * _CHUNK, sg % 4)
            else:
                retire_writeout(t, s_rel * _CHUNK, s % 4)

    for b in range(_NBUF):
        pltpu.make_async_copy(rows[b], out.at[pl.ds(0, _CHUNK)],
                              ssem[b]).wait()


@functools.partial(
    pl.kernel,
    mesh=plsc.VectorSubcoreMesh(core_axis_name="c", subcore_axis_name="s"),
    out_type=jax.ShapeDtypeStruct((_NUM_TABLES * _B, _D), jnp.float32),
    scratch_types=[
        pltpu.VMEM((_NUM_TABLES, _B_PER_W), jnp.int32),
        pltpu.VMEM((_CHUNK, _D), jnp.float32),
        pltpu.VMEM((_CHUNK, _D), jnp.float32),
        pltpu.VMEM((_CHUNK, _D), jnp.float32),
        pltpu.VMEM((_CHUNK, _D), jnp.float32),
        pltpu.SemaphoreType.DMA,
        pltpu.SemaphoreType.DMA,
        pltpu.SemaphoreType.DMA,
        pltpu.SemaphoreType.DMA,
        pltpu.SemaphoreType.DMA,
        pltpu.SemaphoreType.DMA,
        pltpu.SemaphoreType.DMA,
        pltpu.SemaphoreType.DMA,
    ],
)
def _grouped_embedding(*refs):
    _grouped_embedding_body(*refs)


def kernel(values_0, values_1, values_2, values_3,
           weight_0, weight_1, weight_2, weight_3):
    return _grouped_embedding(values_0, values_1, values_2, values_3,
                              weight_0, weight_1, weight_2, weight_3)
